# Initial kernel scaffold; baseline (speedup 1.0000x reference)
import jax
import jax.numpy as jnp
from jax.experimental import pallas as pl


def kernel(x, edge_index, batch, W1, a_src1, a_dst1, b1, W2, a_src2, a_dst2, b2, W3, a_src3, a_dst3, b3, Wl, bl, Wl2, bl2):
    raise NotImplementedError("write your pallas kernel here")


# speedup vs baseline: 1.0930x; 1.0930x over previous
import functools

import jax
import jax.numpy as jnp
from jax.experimental import pallas as pl
from jax.experimental.pallas import tpu as pltpu

N = 10000
E = 320000
HEADS = 5
N_GRAPHS = 100

NPAD = 10240
BN = 256


def _mm_kernel(x_ref, w_ref, as_ref, ad_ref, h_ref, als_ref, ald_ref):
    h = jnp.dot(x_ref[...], w_ref[...], preferred_element_type=jnp.float32)
    h_ref[...] = h
    als_ref[...] = jnp.dot(h, as_ref[...], preferred_element_type=jnp.float32)
    ald_ref[...] = jnp.dot(h, ad_ref[...], preferred_element_type=jnp.float32)


def _proj(x, W, A_s, A_d):
    K = x.shape[1]
    D = W.shape[1]
    grid = (NPAD // BN,)
    return pl.pallas_call(
        _mm_kernel,
        grid=grid,
        in_specs=[
            pl.BlockSpec((BN, K), lambda i: (i, 0)),
            pl.BlockSpec((K, D), lambda i: (0, 0)),
            pl.BlockSpec((D, HEADS), lambda i: (0, 0)),
            pl.BlockSpec((D, HEADS), lambda i: (0, 0)),
        ],
        out_specs=[
            pl.BlockSpec((BN, D), lambda i: (i, 0)),
            pl.BlockSpec((BN, HEADS), lambda i: (i, 0)),
            pl.BlockSpec((BN, HEADS), lambda i: (i, 0)),
        ],
        out_shape=[
            jax.ShapeDtypeStruct((NPAD, D), jnp.float32),
            jax.ShapeDtypeStruct((NPAD, HEADS), jnp.float32),
            jax.ShapeDtypeStruct((NPAD, HEADS), jnp.float32),
        ],
    )(x, W, A_s, A_d)


def _head_kernel(h_ref, batch_ref, wl_ref, bl_ref, wl2_ref, bl2_ref, o_ref):
    b = batch_ref[...]
    gids = jax.lax.broadcasted_iota(jnp.int32, (N_GRAPHS, NPAD), 0)
    oh = jnp.where(b == gids, 1.0, 0.0)
    g = jnp.dot(oh, h_ref[...], preferred_element_type=jnp.float32)
    g = jax.nn.relu(jnp.dot(g, wl_ref[...], preferred_element_type=jnp.float32)
                    + bl_ref[...])
    g = jnp.dot(g, wl2_ref[...], preferred_element_type=jnp.float32) + bl2_ref[...]
    o_ref[...] = jax.nn.sigmoid(g)


def _head(h3, batch_pad, Wl, bl, Wl2, bl2):
    OUT = Wl2.shape[1]
    return pl.pallas_call(
        _head_kernel,
        in_specs=[
            pl.BlockSpec((NPAD, h3.shape[1]), lambda: (0, 0)),
            pl.BlockSpec((1, NPAD), lambda: (0, 0)),
            pl.BlockSpec(Wl.shape, lambda: (0, 0)),
            pl.BlockSpec((1, Wl.shape[1]), lambda: (0, 0)),
            pl.BlockSpec(Wl2.shape, lambda: (0, 0)),
            pl.BlockSpec((1, OUT), lambda: (0, 0)),
        ],
        out_specs=pl.BlockSpec((N_GRAPHS, OUT), lambda: (0, 0)),
        out_shape=jax.ShapeDtypeStruct((N_GRAPHS, OUT), jnp.float32),
    )(h3, batch_pad, Wl, bl.reshape(1, -1), Wl2, bl2.reshape(1, -1))


def _aggregate(h, als, ald, src, dst, heads, dh, b):
    alpha = jax.nn.leaky_relu(als[src] + ald[dst], negative_slope=0.2)
    ex = jnp.exp(alpha)
    den = jax.ops.segment_sum(ex, dst, num_segments=NPAD)
    a = ex / den[dst]
    hsv = h.reshape(NPAD, heads, dh)
    out = jax.ops.segment_sum(hsv[src] * a[:, :, None], dst, num_segments=NPAD)
    return jax.nn.relu(out.reshape(NPAD, heads * dh) + b)


def kernel(x, edge_index, batch, W1, a_src1, a_dst1, b1, W2, a_src2, a_dst2,
           b2, W3, a_src3, a_dst3, b3, Wl, bl, Wl2, bl2):
    xp = jnp.pad(x, ((0, NPAD - N), (0, 0)))
    loop = jnp.arange(N, dtype=edge_index.dtype)
    src = jnp.concatenate([edge_index[0], loop])
    dst = jnp.concatenate([edge_index[1], loop])
    batch_pad = jnp.pad(batch, (0, NPAD - N), constant_values=N_GRAPHS)

    def mk_attn(a_s, a_d, heads, dh):
        A_s = jnp.zeros((heads * dh, heads), jnp.float32)
        A_d = jnp.zeros((heads * dh, heads), jnp.float32)
        rows = jnp.arange(heads * dh)
        cols = rows // dh
        A_s = A_s.at[rows, cols].set(a_s.reshape(-1))
        A_d = A_d.at[rows, cols].set(a_d.reshape(-1))
        return A_s, A_d

    layers = [
        (W1, a_src1, a_dst1, b1, HEADS, 64),
        (W2, a_src2, a_dst2, b2, HEADS, 96),
        (W3, a_src3, a_dst3, b3, 1, 32),
    ]
    h = xp
    for (W, a_s, a_d, b, heads, dh) in layers:
        A_s, A_d = mk_attn(a_s, a_d, heads, dh)
        Wp = jnp.pad(W, ((0, 0), (0, 0)))
        A_sp = jnp.pad(A_s, ((0, 0), (0, HEADS - heads)))
        A_dp = jnp.pad(A_d, ((0, 0), (0, HEADS - heads)))
        hw, als, ald = _proj(h, Wp, A_sp, A_dp)
        h = _aggregate(hw, als[:, :heads], ald[:, :heads], src, dst, heads, dh, b)

    return _head(h, batch_pad.reshape(1, NPAD).astype(jnp.int32),
                 Wl, bl, Wl2, bl2)
